# Initial kernel scaffold; baseline (speedup 1.0000x reference)
import jax
import jax.numpy as jnp
from jax.experimental import pallas as pl


def kernel(inputs, pos_table):
    raise NotImplementedError("write your pallas kernel here")


# speedup vs baseline: 1.6779x; 1.6779x over previous
import jax
import jax.numpy as jnp
from jax.experimental import pallas as pl

_BLOCK_SEQ = 512


def _add_kernel(x_ref, t_ref, o_ref):
    o_ref[...] = x_ref[...] + t_ref[...]


def kernel(inputs, pos_table):
    batch, seq_len, out_dim = inputs.shape
    num_seq_blocks = seq_len // _BLOCK_SEQ
    return pl.pallas_call(
        _add_kernel,
        grid=(num_seq_blocks, batch),
        in_specs=[
            pl.BlockSpec((1, _BLOCK_SEQ, out_dim), lambda s, b: (b, s, 0)),
            pl.BlockSpec((_BLOCK_SEQ, out_dim), lambda s, b: (s, 0)),
        ],
        out_specs=pl.BlockSpec((1, _BLOCK_SEQ, out_dim), lambda s, b: (b, s, 0)),
        out_shape=jax.ShapeDtypeStruct(inputs.shape, inputs.dtype),
    )(inputs, pos_table)


# speedup vs baseline: 1.8413x; 1.0974x over previous
import jax
import jax.numpy as jnp
from jax.experimental import pallas as pl

_BLOCK_SEQ = 1024


def _add_kernel(x_ref, t_ref, o_ref):
    o_ref[...] = x_ref[...] + t_ref[...]


def kernel(inputs, pos_table):
    batch, seq_len, out_dim = inputs.shape
    num_seq_blocks = seq_len // _BLOCK_SEQ
    return pl.pallas_call(
        _add_kernel,
        grid=(num_seq_blocks, batch),
        in_specs=[
            pl.BlockSpec((1, _BLOCK_SEQ, out_dim), lambda s, b: (b, s, 0)),
            pl.BlockSpec((_BLOCK_SEQ, out_dim), lambda s, b: (s, 0)),
        ],
        out_specs=pl.BlockSpec((1, _BLOCK_SEQ, out_dim), lambda s, b: (b, s, 0)),
        out_shape=jax.ShapeDtypeStruct(inputs.shape, inputs.dtype),
    )(inputs, pos_table)


# speedup vs baseline: 1.9713x; 1.0706x over previous
import jax
import jax.numpy as jnp
from jax.experimental import pallas as pl

_BLOCK_SEQ = 2048


def _add_kernel(x_ref, t_ref, o_ref):
    o_ref[...] = x_ref[...] + t_ref[...]


def kernel(inputs, pos_table):
    batch, seq_len, out_dim = inputs.shape
    num_seq_blocks = seq_len // _BLOCK_SEQ
    return pl.pallas_call(
        _add_kernel,
        grid=(num_seq_blocks, batch),
        in_specs=[
            pl.BlockSpec((1, _BLOCK_SEQ, out_dim), lambda s, b: (b, s, 0)),
            pl.BlockSpec((_BLOCK_SEQ, out_dim), lambda s, b: (s, 0)),
        ],
        out_specs=pl.BlockSpec((1, _BLOCK_SEQ, out_dim), lambda s, b: (b, s, 0)),
        out_shape=jax.ShapeDtypeStruct(inputs.shape, inputs.dtype),
    )(inputs, pos_table)
